# Initial kernel scaffold; baseline (speedup 1.0000x reference)
import jax
import jax.numpy as jnp
from jax.experimental import pallas as pl


def kernel(x, Wr, br, W1, b1, W2, b2):
    raise NotImplementedError("write your pallas kernel here")


# speedup vs baseline: 2.7809x; 2.7809x over previous
import functools

import jax
import jax.numpy as jnp
from jax.experimental import pallas as pl
from jax.experimental.pallas import tpu as pltpu

D_MODEL = 1024
D_FF = 4096
N_EXP = 8
FN = 512
NJ = D_FF // FN


def _router_body(x_ref, wr_ref, br_ref, gates_ref):
    logits = jnp.dot(x_ref[...], wr_ref[...], preferred_element_type=jnp.float32)
    logits = logits + br_ref[...]
    T = logits.shape[0]
    idx = jax.lax.broadcasted_iota(jnp.int32, (T, N_EXP), 1)
    m1 = jnp.max(logits, axis=-1, keepdims=True)
    a1 = jnp.min(jnp.where(logits == m1, idx, N_EXP), axis=-1, keepdims=True)
    l2 = jnp.where(idx == a1, -jnp.inf, logits)
    m2 = jnp.max(l2, axis=-1, keepdims=True)
    a2 = jnp.min(jnp.where(l2 == m2, idx, N_EXP), axis=-1, keepdims=True)
    sel = (idx == a1) | (idx == a2)
    e = jnp.where(sel, jnp.exp(logits - m1), 0.0)
    gates_ref[...] = e / jnp.sum(e, axis=-1, keepdims=True)


def _ffn_body(x_ref, w1_ref, b1_ref, w2_ref, b2_ref, g_ref, out_ref, acc_ref):
    ei = pl.program_id(0)
    j = pl.program_id(1)

    @pl.when((ei == 0) & (j == 0))
    def _():
        acc_ref[...] = jnp.zeros_like(acc_ref)

    h = jnp.dot(x_ref[...], w1_ref[0], preferred_element_type=jnp.float32)
    h = h + b1_ref[0]
    h = 0.5 * h * (1.0 + jax.lax.erf(h * 0.7071067811865476))
    part = jnp.dot(h, w2_ref[0], preferred_element_type=jnp.float32)
    part = jnp.where(j == 0, part + b2_ref[0], part)
    lane = jax.lax.broadcasted_iota(jnp.int32, g_ref.shape, 1)
    gate_col = jnp.sum(jnp.where(lane == ei, g_ref[...], 0.0), axis=-1,
                       keepdims=True)
    acc_ref[...] += gate_col * part

    @pl.when((ei == N_EXP - 1) & (j == NJ - 1))
    def _():
        out_ref[...] = acc_ref[...]


def kernel(x, Wr, br, W1, b1, W2, b2):
    bsz, seq, d_model = x.shape
    xf = x.reshape(-1, d_model)
    T = xf.shape[0]

    gates = pl.pallas_call(
        _router_body,
        out_shape=jax.ShapeDtypeStruct((T, N_EXP), jnp.float32),
    )(xf, Wr, br.reshape(1, N_EXP))

    out = pl.pallas_call(
        _ffn_body,
        grid=(N_EXP, NJ),
        in_specs=[
            pl.BlockSpec((T, D_MODEL), lambda e, j: (0, 0)),
            pl.BlockSpec((1, D_MODEL, FN), lambda e, j: (e, 0, j)),
            pl.BlockSpec((1, 1, FN), lambda e, j: (e * NJ + j, 0, 0)),
            pl.BlockSpec((1, FN, D_MODEL), lambda e, j: (e, j, 0)),
            pl.BlockSpec((1, 1, D_MODEL), lambda e, j: (e, 0, 0)),
            pl.BlockSpec((T, N_EXP), lambda e, j: (0, 0)),
        ],
        out_specs=pl.BlockSpec((T, D_MODEL), lambda e, j: (0, 0)),
        out_shape=jax.ShapeDtypeStruct((T, D_MODEL), jnp.float32),
        scratch_shapes=[pltpu.VMEM((T, D_MODEL), jnp.float32)],
    )(xf, W1, b1.reshape(N_EXP * NJ, 1, FN), W2,
      b2.reshape(N_EXP, 1, D_MODEL), gates)

    return out.reshape(bsz, seq, d_model)
